# Initial kernel scaffold; baseline (speedup 1.0000x reference)
import jax
import jax.numpy as jnp
from jax.experimental import pallas as pl


def kernel(x, W1, b1, W2, b2, W3, b3):
    raise NotImplementedError("write your pallas kernel here")


# speedup vs baseline: 4609.6932x; 4609.6932x over previous
import jax
import jax.numpy as jnp
from jax.experimental import pallas as pl
from jax.experimental.pallas import tpu as pltpu

_HORIZON = 10


def _ode_body(n_first, n_total, hid):
    def body(y0_ref, w1_ref, b1_ref, w2_ref, b2_ref, w3_ref, b3_ref,
             dts_ref, out_ref):
        row = jax.lax.broadcasted_iota(jnp.int32, (n_total, 1), 0)
        is_first = row < n_first

        def agg(hw):
            m = jnp.mean(hw[:n_first, :], axis=0, keepdims=True)
            return jnp.where(is_first, m, hw)

        w1 = w1_ref[:, :]
        b1 = b1_ref[:, :]
        w2 = w2_ref[:, :]
        b2 = b2_ref[:, :]
        w3 = w3_ref[:, :]
        b3 = b3_ref[:, :]

        def f(y):
            hw = y * w1
            h = jnp.tanh(agg(hw) + b1)
            hw = jnp.dot(h, w2, preferred_element_type=jnp.float32)
            h = jnp.tanh(agg(hw) + b2)
            hw = jnp.dot(h, w3, preferred_element_type=jnp.float32)
            return agg(hw) + b3

        y = y0_ref[:, :]
        out_ref[:, 0:1] = y
        for i in range(_HORIZON - 1):
            dt = dts_ref[i]
            k1 = f(y)
            k2 = f(y + dt * k1 / 3.0)
            k3 = f(y + dt * (k2 - k1 / 3.0))
            k4 = f(y + dt * (k1 - k2 + k3))
            y = y + dt * (k1 + 3.0 * (k2 + k3) + k4) / 8.0
            out_ref[:, i + 1:i + 2] = y

    return body


def kernel(x, W1, b1, W2, b2, W3, b3):
    Bx, Nx, Tx = x.shape
    n_total = Bx * Nx
    hid = W1.shape[1]

    y0 = x[:, :, -1].reshape(n_total, 1)
    ts = jnp.linspace(0.0, float(_HORIZON), _HORIZON)
    dts = ts[1:] - ts[:-1]

    out = pl.pallas_call(
        _ode_body(Nx, n_total, hid),
        out_shape=jax.ShapeDtypeStruct((n_total, _HORIZON), jnp.float32),
        in_specs=[
            pl.BlockSpec(memory_space=pltpu.VMEM),
            pl.BlockSpec(memory_space=pltpu.VMEM),
            pl.BlockSpec(memory_space=pltpu.VMEM),
            pl.BlockSpec(memory_space=pltpu.VMEM),
            pl.BlockSpec(memory_space=pltpu.VMEM),
            pl.BlockSpec(memory_space=pltpu.VMEM),
            pl.BlockSpec(memory_space=pltpu.VMEM),
            pl.BlockSpec(memory_space=pltpu.SMEM),
        ],
        out_specs=pl.BlockSpec(memory_space=pltpu.VMEM),
    )(y0, W1, b1.reshape(1, hid), W2, b2.reshape(1, hid),
      W3, b3.reshape(1, 1), dts)

    return out.reshape(Bx, Nx, _HORIZON)


# speedup vs baseline: 6226.5241x; 1.3507x over previous
import jax
import jax.numpy as jnp
from jax.experimental import pallas as pl
from jax.experimental.pallas import tpu as pltpu

_HORIZON = 10


def _ode_body(n_first, n_rest, hid):
    S = 1 + n_rest

    def body(x0_ref, yrest_ref, w1_ref, b1_ref, w2_ref, b2_ref, w3_ref,
             b3_ref, dts_ref, out0_ref, out1_ref):
        w1 = w1_ref[:, :]
        b1 = b1_ref[:, :]
        w2 = w2_ref[:, :]
        b2 = b2_ref[:, :]
        w3 = w3_ref[:, :]
        b3 = b3_ref[:, :]

        def g(s):
            h = jnp.broadcast_to(w1, (hid, S)) * jnp.broadcast_to(s, (hid, S))
            h = jnp.tanh(h + jnp.broadcast_to(b1, (hid, S)))
            h = jnp.dot(w2, h, preferred_element_type=jnp.float32)
            h = jnp.tanh(h + jnp.broadcast_to(b2, (hid, S)))
            out = jnp.dot(w3, h, preferred_element_type=jnp.float32)
            return out + jnp.broadcast_to(b3, (1, S))

        x0 = x0_ref[:, :]
        mu0 = jnp.mean(x0, axis=1, keepdims=True)
        s = jnp.concatenate([mu0, yrest_ref[:, :]], axis=1)

        out1_ref[0:1, :] = s
        mus = [s[0:1, 0:1]]
        for i in range(_HORIZON - 1):
            dt = dts_ref[i]
            k1 = g(s)
            k2 = g(s + dt * k1 / 3.0)
            k3 = g(s + dt * (k2 - k1 / 3.0))
            k4 = g(s + dt * (k1 - k2 + k3))
            s = s + dt * (k1 + 3.0 * (k2 + k3) + k4) / 8.0
            out1_ref[i + 1:i + 2, :] = s
            mus.append(s[0:1, 0:1])

        offs = jnp.concatenate(mus, axis=0) - mus[0]
        out0_ref[:, :] = (jnp.broadcast_to(offs, (_HORIZON, n_first)) +
                          jnp.broadcast_to(x0, (_HORIZON, n_first)))

    return body


def kernel(x, W1, b1, W2, b2, W3, b3):
    Bx, Nx, Tx = x.shape
    hid = W1.shape[1]
    n_rest = (Bx - 1) * Nx

    last = x[:, :, -1]
    x0row = last[0].reshape(1, Nx)
    yrest = last[1:].reshape(1, n_rest)
    ts = jnp.linspace(0.0, float(_HORIZON), _HORIZON)
    dts = ts[1:] - ts[:-1]

    out0, out1 = pl.pallas_call(
        _ode_body(Nx, n_rest, hid),
        out_shape=(
            jax.ShapeDtypeStruct((_HORIZON, Nx), jnp.float32),
            jax.ShapeDtypeStruct((_HORIZON, 1 + n_rest), jnp.float32),
        ),
        in_specs=[pl.BlockSpec(memory_space=pltpu.VMEM)] * 8 +
                 [pl.BlockSpec(memory_space=pltpu.SMEM)],
        out_specs=(pl.BlockSpec(memory_space=pltpu.VMEM),
                   pl.BlockSpec(memory_space=pltpu.VMEM)),
    )(x0row, yrest, W1.reshape(hid, 1), b1.reshape(hid, 1), W2.T,
      b2.reshape(hid, 1), W3.reshape(1, hid), b3.reshape(1, 1), dts)

    rest = out1[:, 1:].T.reshape(Bx - 1, Nx, _HORIZON)
    return jnp.concatenate([out0.T.reshape(1, Nx, _HORIZON), rest], axis=0)
